# Initial kernel scaffold; baseline (speedup 1.0000x reference)
import jax
import jax.numpy as jnp
from jax.experimental import pallas as pl


def kernel(inputs, table):
    raise NotImplementedError("write your pallas kernel here")


# speedup vs baseline: 3.7007x; 3.7007x over previous
import functools

import jax
import jax.numpy as jnp
from jax import lax
from jax.experimental import pallas as pl
from jax.experimental.pallas import tpu as pltpu
from jax.experimental.pallas import tpu_sc as plsc


def _make_sc_broadcast(B, S, D, dtype):
    info = plsc.get_sparse_core_info()
    NC, NS = info.num_cores, info.num_subcores
    NW = NC * NS
    rows_per_w = S // NW
    CH = 64
    n_ch = rows_per_w // CH
    mesh = plsc.VectorSubcoreMesh(core_axis_name="c", subcore_axis_name="s")

    @functools.partial(
        pl.kernel,
        mesh=mesh,
        out_type=jax.ShapeDtypeStruct((B, S, D), dtype),
        scratch_types=[
            pltpu.VMEM((CH, D), dtype),
            pltpu.SemaphoreType.DMA,
        ],
    )
    def k(table_hbm, out_hbm, buf, sem):
        wid = lax.axis_index("s") * NC + lax.axis_index("c")
        base = wid * rows_per_w
        for c in range(n_ch):
            lo = base + c * CH
            pltpu.sync_copy(table_hbm.at[pl.ds(lo, CH)], buf)
            for b in range(B):
                pltpu.sync_copy(buf, out_hbm.at[b, pl.ds(lo, CH)])

    return k


def kernel(inputs, table):
    B, S = inputs.shape
    V, D = table.shape
    return _make_sc_broadcast(B, S, D, table.dtype)(table)


# speedup vs baseline: 3.8283x; 1.0345x over previous
import functools

import jax
import jax.numpy as jnp
from jax import lax
from jax.experimental import pallas as pl
from jax.experimental.pallas import tpu as pltpu
from jax.experimental.pallas import tpu_sc as plsc


def _make_sc_broadcast(B, S, D, dtype):
    info = plsc.get_sparse_core_info()
    NC, NS = info.num_cores, info.num_subcores
    NW = NC * NS
    rows_per_w = S // NW
    CH = 64
    n_ch = rows_per_w // CH
    mesh = plsc.VectorSubcoreMesh(core_axis_name="c", subcore_axis_name="s")

    NBUF = 2

    @functools.partial(
        pl.kernel,
        mesh=mesh,
        out_type=jax.ShapeDtypeStruct((B, S, D), dtype),
        scratch_types=[
            pltpu.VMEM((CH, D), dtype),
            pltpu.VMEM((CH, D), dtype),
            pltpu.SemaphoreType.DMA,
            pltpu.SemaphoreType.DMA,
            pltpu.SemaphoreType.DMA,
            pltpu.SemaphoreType.DMA,
        ],
    )
    def k(table_hbm, out_hbm, buf0, buf1, in0, in1, out0, out1):
        wid = lax.axis_index("s") * NC + lax.axis_index("c")
        base = wid * rows_per_w
        bufs, in_sems, out_sems = (buf0, buf1), (in0, in1), (out0, out1)
        loads = {}
        stores = {}

        def start_load(c):
            lo = base + c * CH
            loads[c] = pltpu.async_copy(
                table_hbm.at[pl.ds(lo, CH)], bufs[c % NBUF], in_sems[c % NBUF])

        def fire_stores(c):
            lo = base + c * CH
            stores[c] = [
                pltpu.async_copy(
                    bufs[c % NBUF], out_hbm.at[b, pl.ds(lo, CH)],
                    out_sems[c % NBUF])
                for b in range(B)
            ]

        for c in range(min(NBUF, n_ch)):
            start_load(c)
        for c in range(n_ch):
            if c >= NBUF:
                for h in stores.pop(c - NBUF):
                    h.wait()
                start_load(c)
            loads.pop(c).wait()
            fire_stores(c)
        for c in sorted(stores):
            for h in stores.pop(c):
                h.wait()

    return k


def kernel(inputs, table):
    B, S = inputs.shape
    V, D = table.shape
    return _make_sc_broadcast(B, S, D, table.dtype)(table)
